# Initial kernel scaffold; baseline (speedup 1.0000x reference)
import jax
import jax.numpy as jnp
from jax.experimental import pallas as pl


def kernel(x, W):
    raise NotImplementedError("write your pallas kernel here")


# speedup vs baseline: 2.6467x; 2.6467x over previous
import jax
import jax.numpy as jnp
from jax.experimental import pallas as pl

N, D = 100000, 128
BS = 1000
G = 16


def _body(x_ref, w_ref, out_ref, wout_ref):
    pid = pl.program_id(0)
    w = w_ref[...]
    wout_ref[...] = w
    out_ref[...] = x_ref[...] + w

    @pl.when(pid < 4)
    def _():
        eps = (pid.astype(jnp.float32) + 1.0) * 0.5
        g = w[:G, :]
        norm = jnp.sqrt(jnp.sum(g * g, axis=1, keepdims=True))
        gn = g * (eps / jnp.maximum(norm, eps))
        wout_ref[:G, :] = gn
        out_ref[:G, :] = x_ref[:G, :] + gn


def kernel(x, W):
    out, Wout = pl.pallas_call(
        _body,
        grid=(N // BS,),
        in_specs=[
            pl.BlockSpec((BS, D), lambda i: (i, 0)),
            pl.BlockSpec((BS, D), lambda i: (i, 0)),
        ],
        out_specs=[
            pl.BlockSpec((BS, D), lambda i: (i, 0)),
            pl.BlockSpec((BS, D), lambda i: (i, 0)),
        ],
        out_shape=[
            jax.ShapeDtypeStruct((N, D), jnp.float32),
            jax.ShapeDtypeStruct((N, D), jnp.float32),
        ],
    )(x, W)
    return (out, Wout)


# speedup vs baseline: 3.8029x; 1.4368x over previous
import jax
import jax.numpy as jnp
from jax.experimental import pallas as pl

N, D = 100000, 128
BS = 2000
G = 16


def _body(x_ref, w_ref, out_ref, wout_ref):
    pid = pl.program_id(0)
    w = w_ref[...]
    wout_ref[...] = w
    out_ref[...] = x_ref[...] + w

    for g in range(4):
        blk, off = (g * 1000) // BS, (g * 1000) % BS
        eps = 0.5 * (g + 1)

        @pl.when(pid == blk)
        def _(off=off, eps=eps):
            gw = w[off:off + G, :]
            norm = jnp.sqrt(jnp.sum(gw * gw, axis=1, keepdims=True))
            gn = gw * (eps / jnp.maximum(norm, eps))
            wout_ref[off:off + G, :] = gn
            out_ref[off:off + G, :] = x_ref[off:off + G, :] + gn


def kernel(x, W):
    out, Wout = pl.pallas_call(
        _body,
        grid=(N // BS,),
        in_specs=[
            pl.BlockSpec((BS, D), lambda i: (i, 0)),
            pl.BlockSpec((BS, D), lambda i: (i, 0)),
        ],
        out_specs=[
            pl.BlockSpec((BS, D), lambda i: (i, 0)),
            pl.BlockSpec((BS, D), lambda i: (i, 0)),
        ],
        out_shape=[
            jax.ShapeDtypeStruct((N, D), jnp.float32),
            jax.ShapeDtypeStruct((N, D), jnp.float32),
        ],
    )(x, W)
    return (out, Wout)


# speedup vs baseline: 4.1653x; 1.0953x over previous
import jax
import jax.numpy as jnp
from jax.experimental import pallas as pl

N, D = 100000, 128
BS = 4000
G = 16


def _body(x_ref, w_ref, out_ref, wout_ref):
    pid = pl.program_id(0)
    w = w_ref[...]
    wout_ref[...] = w
    out_ref[...] = x_ref[...] + w

    for g in range(4):
        blk, off = (g * 1000) // BS, (g * 1000) % BS
        eps = 0.5 * (g + 1)

        @pl.when(pid == blk)
        def _(off=off, eps=eps):
            gw = w[off:off + G, :]
            norm = jnp.sqrt(jnp.sum(gw * gw, axis=1, keepdims=True))
            gn = gw * (eps / jnp.maximum(norm, eps))
            wout_ref[off:off + G, :] = gn
            out_ref[off:off + G, :] = x_ref[off:off + G, :] + gn


def kernel(x, W):
    out, Wout = pl.pallas_call(
        _body,
        grid=(N // BS,),
        in_specs=[
            pl.BlockSpec((BS, D), lambda i: (i, 0)),
            pl.BlockSpec((BS, D), lambda i: (i, 0)),
        ],
        out_specs=[
            pl.BlockSpec((BS, D), lambda i: (i, 0)),
            pl.BlockSpec((BS, D), lambda i: (i, 0)),
        ],
        out_shape=[
            jax.ShapeDtypeStruct((N, D), jnp.float32),
            jax.ShapeDtypeStruct((N, D), jnp.float32),
        ],
    )(x, W)
    return (out, Wout)


# speedup vs baseline: 4.2885x; 1.0296x over previous
import jax
import jax.numpy as jnp
from jax.experimental import pallas as pl

N, D = 100000, 128
BS = 10000
G = 16


def _body(x_ref, w_ref, out_ref, wout_ref):
    pid = pl.program_id(0)
    w = w_ref[...]
    wout_ref[...] = w
    out_ref[...] = x_ref[...] + w

    for g in range(4):
        blk, off = (g * 1000) // BS, (g * 1000) % BS
        eps = 0.5 * (g + 1)

        @pl.when(pid == blk)
        def _(off=off, eps=eps):
            gw = w[off:off + G, :]
            norm = jnp.sqrt(jnp.sum(gw * gw, axis=1, keepdims=True))
            gn = gw * (eps / jnp.maximum(norm, eps))
            wout_ref[off:off + G, :] = gn
            out_ref[off:off + G, :] = x_ref[off:off + G, :] + gn


def kernel(x, W):
    out, Wout = pl.pallas_call(
        _body,
        grid=(N // BS,),
        in_specs=[
            pl.BlockSpec((BS, D), lambda i: (i, 0)),
            pl.BlockSpec((BS, D), lambda i: (i, 0)),
        ],
        out_specs=[
            pl.BlockSpec((BS, D), lambda i: (i, 0)),
            pl.BlockSpec((BS, D), lambda i: (i, 0)),
        ],
        out_shape=[
            jax.ShapeDtypeStruct((N, D), jnp.float32),
            jax.ShapeDtypeStruct((N, D), jnp.float32),
        ],
    )(x, W)
    return (out, Wout)


# speedup vs baseline: 4.3085x; 1.0047x over previous
import jax
import jax.numpy as jnp
from jax.experimental import pallas as pl
from jax.experimental.pallas import tpu as pltpu

N, D = 100000, 128
BS = 10000
G = 16


def _body(x_ref, w_ref, out_ref, wout_ref):
    pid = pl.program_id(0)
    w = w_ref[...]
    wout_ref[...] = w
    out_ref[...] = x_ref[...] + w

    for g in range(4):
        blk, off = (g * 1000) // BS, (g * 1000) % BS
        eps = 0.5 * (g + 1)

        @pl.when(pid == blk)
        def _(off=off, eps=eps):
            gw = w[off:off + G, :]
            norm = jnp.sqrt(jnp.sum(gw * gw, axis=1, keepdims=True))
            gn = gw * (eps / jnp.maximum(norm, eps))
            wout_ref[off:off + G, :] = gn
            out_ref[off:off + G, :] = x_ref[off:off + G, :] + gn


def kernel(x, W):
    out, Wout = pl.pallas_call(
        _body,
        grid=(N // BS,),
        in_specs=[
            pl.BlockSpec((BS, D), lambda i: (i, 0)),
            pl.BlockSpec((BS, D), lambda i: (i, 0)),
        ],
        out_specs=[
            pl.BlockSpec((BS, D), lambda i: (i, 0)),
            pl.BlockSpec((BS, D), lambda i: (i, 0)),
        ],
        out_shape=[
            jax.ShapeDtypeStruct((N, D), jnp.float32),
            jax.ShapeDtypeStruct((N, D), jnp.float32),
        ],
        compiler_params=pltpu.CompilerParams(
            dimension_semantics=("parallel",),
        ),
    )(x, W)
    return (out, Wout)
